# baseline (speedup 1.0000x reference)
import jax
import jax.numpy as jnp
from jax import lax
from jax.experimental import pallas as pl
from jax.experimental.pallas import tpu as pltpu, tpu_sc as plsc

N_NODES = 10000
N_EDGES = 160000
D_FEAT = 256

NC = 2
NS = 16
DH = D_FEAT // NC
EPT = N_EDGES // NS
CHUNK = 128
NCHUNK = EPT // CHUNK
TAIL = EPT - NCHUNK * CHUNK
RPT = N_NODES // NS
RBLK = 25
NRB = RPT // RBLK


def _scale(buf, valsb, n_edges):
    for jo in range(0, n_edges, 16):
        vblk = valsb[pl.ds(jo, 16)]
        for ji in range(16):
            j = jo + ji
            vv = vblk[ji]
            for g in range(DH // 16):
                sl = pl.ds(g * 16, 16)
                buf[j, sl] = buf[j, sl] * vv


def _gcn_body(emb_hbm, col_hbm, row_hbm, vals_hbm, out_hbm,
              col_v, rowi0, rowi1, vals0, vals1, rows0, rows1,
              rowi_t, vals_t, obuf, acc, sem0, sem1):
    c = lax.axis_index("c")
    s = lax.axis_index("s")

    def _zero(i, _):
        for g in range(DH // 16):
            obuf[i, pl.ds(g * 16, 16)] = jnp.zeros((16,), jnp.float32)
        return 0
    lax.fori_loop(0, RBLK, _zero, 0)
    for b in range(NRB):
        pltpu.sync_copy(obuf, acc.at[pl.ds(s * RPT + b * RBLK, RBLK), :])

    ebase = s * EPT
    pltpu.sync_copy(col_hbm.at[pl.ds(ebase, EPT)], col_v)
    plsc.subcore_barrier()

    emb_c = emb_hbm.at[c]

    def _gather(ch, buf, rowi, valsb, sem):
        base = ebase + ch * CHUNK
        idx = col_v.at[pl.ds(ch * CHUNK, CHUNK)]
        pltpu.async_copy(emb_c.at[idx], buf, sem)
        pltpu.async_copy(row_hbm.at[pl.ds(base, CHUNK)], rowi, sem)
        pltpu.async_copy(vals_hbm.at[pl.ds(base, CHUNK)], valsb, sem)

    def _wait(buf, rowi, valsb, sem):
        pltpu.make_async_copy(emb_c.at[col_v.at[pl.ds(0, CHUNK)]], buf,
                              sem).wait()
        pltpu.make_async_copy(row_hbm.at[pl.ds(0, CHUNK)], rowi, sem).wait()
        pltpu.make_async_copy(vals_hbm.at[pl.ds(0, CHUNK)], valsb, sem).wait()

    _gather(0, rows0, rowi0, vals0, sem0)
    _gather(1, rows1, rowi1, vals1, sem1)
    NPAIR = NCHUNK // 2

    def _pair(i, _):
        ch0 = 2 * i
        _wait(rows0, rowi0, vals0, sem0)
        _scale(rows0, vals0, CHUNK)
        pltpu.sync_copy(rows0, acc.at[rowi0], add=True)

        @pl.when(i < NPAIR - 1)
        def _():
            _gather(ch0 + 2, rows0, rowi0, vals0, sem0)
        _wait(rows1, rowi1, vals1, sem1)
        _scale(rows1, vals1, CHUNK)
        pltpu.sync_copy(rows1, acc.at[rowi1], add=True)

        @pl.when(i < NPAIR - 1)
        def _():
            _gather(ch0 + 3, rows1, rowi1, vals1, sem1)
        return 0
    lax.fori_loop(0, NPAIR, _pair, 0)

    tbase = ebase + NCHUNK * CHUNK
    tidx = col_v.at[pl.ds(NCHUNK * CHUNK, TAIL)]
    trows = rows0.at[pl.ds(0, TAIL), :]
    pltpu.async_copy(emb_c.at[tidx], trows, sem0)
    pltpu.async_copy(row_hbm.at[pl.ds(tbase, TAIL)], rowi_t, sem0)
    pltpu.async_copy(vals_hbm.at[pl.ds(tbase, TAIL)], vals_t, sem0)
    pltpu.make_async_copy(emb_c.at[tidx], trows, sem0).wait()
    pltpu.make_async_copy(row_hbm.at[pl.ds(tbase, TAIL)], rowi_t, sem0).wait()
    pltpu.make_async_copy(vals_hbm.at[pl.ds(tbase, TAIL)], vals_t, sem0).wait()
    _scale(rows0, vals_t, TAIL)
    pltpu.sync_copy(trows, acc.at[rowi_t], add=True)
    plsc.subcore_barrier()

    for b in range(NRB):
        r0 = s * RPT + b * RBLK
        pltpu.sync_copy(acc.at[pl.ds(r0, RBLK), :], obuf)

        def _lrelu(i, _):
            for g in range(DH // 16):
                sl = pl.ds(g * 16, 16)
                x = obuf[i, sl]
                obuf[i, sl] = jnp.maximum(x, x * 0.5)
            return 0
        lax.fori_loop(0, RBLK, _lrelu, 0)
        pltpu.sync_copy(obuf, out_hbm.at[c, pl.ds(r0, RBLK), :])


def kernel(edge_index, edge_vals, embeds):
    emb_split = embeds.reshape(N_NODES, NC, DH).transpose(1, 0, 2)
    col = edge_index[1]
    row = edge_index[0]

    k = pl.kernel(
        _gcn_body,
        out_type=jax.ShapeDtypeStruct((NC, N_NODES, DH), jnp.float32),
        mesh=plsc.VectorSubcoreMesh(core_axis_name="c", subcore_axis_name="s"),
        compiler_params=pltpu.CompilerParams(use_tc_tiling_on_sc=False),
        scratch_types=[
            pltpu.VMEM((EPT,), jnp.int32),
            pltpu.VMEM((CHUNK,), jnp.int32),
            pltpu.VMEM((CHUNK,), jnp.int32),
            pltpu.VMEM((CHUNK,), jnp.float32),
            pltpu.VMEM((CHUNK,), jnp.float32),
            pltpu.VMEM((CHUNK, DH), jnp.float32),
            pltpu.VMEM((CHUNK, DH), jnp.float32),
            pltpu.VMEM((TAIL,), jnp.int32),
            pltpu.VMEM((TAIL,), jnp.float32),
            pltpu.VMEM((RBLK, DH), jnp.float32),
            pltpu.VMEM_SHARED((N_NODES, DH), jnp.float32),
            pltpu.SemaphoreType.DMA,
            pltpu.SemaphoreType.DMA,
        ],
    )
    out = k(emb_split, col, row, edge_vals)
    return out.transpose(1, 0, 2).reshape(N_NODES, D_FEAT)


# speedup vs baseline: 1.3167x; 1.3167x over previous
import jax
import jax.numpy as jnp
from jax import lax
from jax.experimental import pallas as pl
from jax.experimental.pallas import tpu as pltpu, tpu_sc as plsc

N_NODES = 10000
N_EDGES = 160000
D_FEAT = 256

NC = 2
NS = 16
DH = D_FEAT // NC
EPT = N_EDGES // NS
CHUNK = 80
NCHUNK = EPT // CHUNK
RPT = N_NODES // NS
RBLK = 25
NRB = RPT // RBLK


def _gcn_body(emb_hbm, col_hbm, row_hbm, vals_hbm, out_hbm,
              col_v, vals_v, rowi0, rowi1, rows0, rows1, ob0, ob1, acc,
              sem0, sem1, osem0, osem1):
    c = lax.axis_index("c")
    s = lax.axis_index("s")

    ebase = s * EPT
    pltpu.async_copy(col_hbm.at[pl.ds(ebase, EPT)], col_v, sem1)
    pltpu.async_copy(vals_hbm.at[pl.ds(ebase, EPT)], vals_v, sem1)

    def _zset(i, _):
        for g in range(DH // 16):
            ob0[i, pl.ds(g * 16, 16)] = jnp.zeros((16,), jnp.float32)
        return 0
    lax.fori_loop(0, RBLK, _zset, 0)
    for b in range(NRB):
        pltpu.async_copy(ob0, acc.at[pl.ds(s * RPT + b * RBLK, RBLK), :],
                         sem0)
    for b in range(NRB):
        pltpu.make_async_copy(ob0, acc.at[pl.ds(s * RPT, RBLK), :],
                              sem0).wait()
    pltpu.make_async_copy(col_hbm.at[pl.ds(0, EPT)], col_v, sem1).wait()
    pltpu.make_async_copy(vals_hbm.at[pl.ds(0, EPT)], vals_v, sem1).wait()
    plsc.subcore_barrier()

    emb_c = emb_hbm.at[c]

    def _gather(ch, buf, rowi, sem):
        idx = col_v.at[pl.ds(ch * CHUNK, CHUNK)]
        pltpu.async_copy(emb_c.at[idx], buf, sem)
        pltpu.async_copy(row_hbm.at[pl.ds(ebase + ch * CHUNK, CHUNK)],
                         rowi, sem)

    def _wait(buf, rowi, sem):
        pltpu.make_async_copy(emb_c.at[col_v.at[pl.ds(0, CHUNK)]], buf,
                              sem).wait()
        pltpu.make_async_copy(row_hbm.at[pl.ds(0, CHUNK)], rowi, sem).wait()

    def _compute_scatter(ch, buf, rowi):
        for jo in range(0, CHUNK, 16):
            vblk = vals_v[pl.ds(ch * CHUNK + jo, 16)]
            for ji in range(16):
                j = jo + ji
                vv = vblk[ji]
                for g in range(DH // 16):
                    sl = pl.ds(g * 16, 16)
                    buf[j, sl] = buf[j, sl] * vv
        pltpu.sync_copy(buf, acc.at[rowi], add=True)

    _gather(0, rows0, rowi0, sem0)
    _gather(1, rows1, rowi1, sem1)

    def _pair(i, _):
        ch0 = 2 * i
        _wait(rows0, rowi0, sem0)
        _compute_scatter(ch0, rows0, rowi0)
        _gather(ch0 + 2, rows0, rowi0, sem0)
        _wait(rows1, rowi1, sem1)
        _compute_scatter(ch0 + 1, rows1, rowi1)

        @pl.when(i < (NCHUNK - 1) // 2 - 1)
        def _():
            _gather(ch0 + 3, rows1, rowi1, sem1)
        return 0
    lax.fori_loop(0, (NCHUNK - 1) // 2, _pair, 0)

    _wait(rows0, rowi0, sem0)
    _compute_scatter(NCHUNK - 1, rows0, rowi0)
    plsc.subcore_barrier()

    rbase = s * RPT

    def _ain(b, ob, isem):
        pltpu.async_copy(acc.at[pl.ds(rbase + b * RBLK, RBLK), :], ob, isem)

    def _iwait(ob, isem):
        pltpu.make_async_copy(acc.at[pl.ds(rbase, RBLK), :], ob, isem).wait()

    def _aout(b, ob, osem):
        pltpu.async_copy(ob, out_hbm.at[c, pl.ds(rbase + b * RBLK, RBLK), :],
                         osem)

    def _owait(ob, osem):
        pltpu.make_async_copy(ob, out_hbm.at[c, pl.ds(rbase, RBLK), :],
                              osem).wait()

    def _lrelu(ob):
        def body(i, _):
            for g in range(DH // 16):
                sl = pl.ds(g * 16, 16)
                x = ob[i, sl]
                ob[i, sl] = jnp.maximum(x, x * 0.5)
            return 0
        lax.fori_loop(0, RBLK, body, 0)

    _ain(0, ob0, sem0)
    _ain(1, ob1, sem1)

    def _dpair(i, _):
        b0 = 2 * i
        _iwait(ob0, sem0)
        _lrelu(ob0)
        _aout(b0, ob0, osem0)
        _iwait(ob1, sem1)
        _lrelu(ob1)
        _aout(b0 + 1, ob1, osem1)
        _owait(ob0, osem0)
        _ain(b0 + 2, ob0, sem0)

        @pl.when(i < NRB // 2 - 1)
        def _():
            _owait(ob1, osem1)
            _ain(b0 + 3, ob1, sem1)
        return 0
    lax.fori_loop(0, NRB // 2, _dpair, 0)

    _iwait(ob0, sem0)
    _lrelu(ob0)
    _aout(NRB - 1, ob0, osem0)
    _owait(ob0, osem0)
    _owait(ob1, osem1)


def kernel(edge_index, edge_vals, embeds):
    emb_split = embeds.reshape(N_NODES, NC, DH).transpose(1, 0, 2)
    col = edge_index[1]
    row = edge_index[0]

    k = pl.kernel(
        _gcn_body,
        out_type=jax.ShapeDtypeStruct((NC, N_NODES, DH), jnp.float32),
        mesh=plsc.VectorSubcoreMesh(core_axis_name="c", subcore_axis_name="s"),
        compiler_params=pltpu.CompilerParams(use_tc_tiling_on_sc=False),
        scratch_types=[
            pltpu.VMEM((EPT,), jnp.int32),
            pltpu.VMEM((EPT,), jnp.float32),
            pltpu.VMEM((CHUNK,), jnp.int32),
            pltpu.VMEM((CHUNK,), jnp.int32),
            pltpu.VMEM((CHUNK, DH), jnp.float32),
            pltpu.VMEM((CHUNK, DH), jnp.float32),
            pltpu.VMEM((RBLK, DH), jnp.float32),
            pltpu.VMEM((RBLK, DH), jnp.float32),
            pltpu.VMEM_SHARED((N_NODES, DH), jnp.float32),
            pltpu.SemaphoreType.DMA,
            pltpu.SemaphoreType.DMA,
            pltpu.SemaphoreType.DMA,
            pltpu.SemaphoreType.DMA,
        ],
    )
    out = k(emb_split, col, row, edge_vals)
    return out.transpose(1, 0, 2).reshape(N_NODES, D_FEAT)


# speedup vs baseline: 1.3980x; 1.0618x over previous
import jax
import jax.numpy as jnp
from jax import lax
from jax.experimental import pallas as pl
from jax.experimental.pallas import tpu as pltpu, tpu_sc as plsc

N_NODES = 10000
N_EDGES = 160000
D_FEAT = 256

NC = 2
NS = 16
DH = D_FEAT // NC
EPT = N_EDGES // NS
CHUNK = 80
NCHUNK = EPT // CHUNK
RPT = N_NODES // NS
RBLK = 25
NRB = RPT // RBLK


def _gcn_body(emb_hbm, col_hbm, row_hbm, vals_hbm, out_hbm,
              col_v, vals_v, rowi0, rowi1, rows0, rows1, ob0, ob1, acc,
              sem0, sem1, osem0, osem1):
    c = lax.axis_index("c")
    s = lax.axis_index("s")

    ebase = s * EPT
    pltpu.async_copy(col_hbm.at[pl.ds(ebase, EPT)], col_v, sem1)
    pltpu.async_copy(vals_hbm.at[pl.ds(ebase, EPT)], vals_v, sem1)

    def _zset(i, _):
        for g in range(DH // 16):
            ob0[i, pl.ds(g * 16, 16)] = jnp.zeros((16,), jnp.float32)
        return 0
    lax.fori_loop(0, RBLK, _zset, 0)
    for b in range(NRB):
        pltpu.async_copy(ob0, acc.at[pl.ds(s * RPT + b * RBLK, RBLK), :],
                         sem0)
    for b in range(NRB):
        pltpu.make_async_copy(ob0, acc.at[pl.ds(s * RPT, RBLK), :],
                              sem0).wait()
    pltpu.make_async_copy(col_hbm.at[pl.ds(0, EPT)], col_v, sem1).wait()
    pltpu.make_async_copy(vals_hbm.at[pl.ds(0, EPT)], vals_v, sem1).wait()
    plsc.subcore_barrier()

    emb_c = emb_hbm.at[c]

    def _gather(ch, buf, rowi, sem):
        idx = col_v.at[pl.ds(ch * CHUNK, CHUNK)]
        pltpu.async_copy(emb_c.at[idx], buf, sem)
        pltpu.async_copy(row_hbm.at[pl.ds(ebase + ch * CHUNK, CHUNK)],
                         rowi, sem)

    def _wait(buf, rowi, sem):
        pltpu.make_async_copy(emb_c.at[col_v.at[pl.ds(0, CHUNK)]], buf,
                              sem).wait()
        pltpu.make_async_copy(row_hbm.at[pl.ds(0, CHUNK)], rowi, sem).wait()

    def _compute_scatter(ch, buf, rowi):
        for jo in range(0, CHUNK, 16):
            vblk = vals_v[pl.ds(ch * CHUNK + jo, 16)]
            for ji in range(16):
                j = jo + ji
                vv = vblk[ji]
                for g in range(DH // 16):
                    sl = pl.ds(g * 16, 16)
                    buf[j, sl] = buf[j, sl] * vv
        pltpu.sync_copy(buf, acc.at[rowi], add=True)

    _gather(0, rows0, rowi0, sem0)
    _gather(1, rows1, rowi1, sem1)

    def _pair(i, _):
        ch0 = 2 * i
        _wait(rows0, rowi0, sem0)
        _compute_scatter(ch0, rows0, rowi0)
        _gather(ch0 + 2, rows0, rowi0, sem0)
        _wait(rows1, rowi1, sem1)
        _compute_scatter(ch0 + 1, rows1, rowi1)

        @pl.when(i < (NCHUNK - 1) // 2 - 1)
        def _():
            _gather(ch0 + 3, rows1, rowi1, sem1)
        return 0
    lax.fori_loop(0, (NCHUNK - 1) // 2, _pair, 0)

    _wait(rows0, rowi0, sem0)
    _compute_scatter(NCHUNK - 1, rows0, rowi0)
    plsc.subcore_barrier()

    rbase = s * RPT

    def _ain(b, ob, isem):
        pltpu.async_copy(acc.at[pl.ds(rbase + b * RBLK, RBLK), :], ob, isem)

    def _iwait(ob, isem):
        pltpu.make_async_copy(acc.at[pl.ds(rbase, RBLK), :], ob, isem).wait()

    def _aout(b, ob, osem):
        pltpu.async_copy(
            ob,
            out_hbm.at[pl.ds(rbase + b * RBLK, RBLK), pl.ds(c * DH, DH)],
            osem)

    def _owait(ob, osem):
        pltpu.make_async_copy(
            ob, out_hbm.at[pl.ds(rbase, RBLK), pl.ds(c * DH, DH)],
            osem).wait()

    def _lrelu(ob):
        def body(i, _):
            for g in range(DH // 16):
                sl = pl.ds(g * 16, 16)
                x = ob[i, sl]
                ob[i, sl] = jnp.maximum(x, x * 0.5)
            return 0
        lax.fori_loop(0, RBLK, body, 0)

    _ain(0, ob0, sem0)
    _ain(1, ob1, sem1)

    def _dpair(i, _):
        b0 = 2 * i
        _iwait(ob0, sem0)
        _lrelu(ob0)
        _aout(b0, ob0, osem0)
        _iwait(ob1, sem1)
        _lrelu(ob1)
        _aout(b0 + 1, ob1, osem1)
        _owait(ob0, osem0)
        _ain(b0 + 2, ob0, sem0)

        @pl.when(i < NRB // 2 - 1)
        def _():
            _owait(ob1, osem1)
            _ain(b0 + 3, ob1, sem1)
        return 0
    lax.fori_loop(0, NRB // 2, _dpair, 0)

    _iwait(ob0, sem0)
    _lrelu(ob0)
    _aout(NRB - 1, ob0, osem0)
    _owait(ob0, osem0)
    _owait(ob1, osem1)


def kernel(edge_index, edge_vals, embeds):
    emb_split = embeds.reshape(N_NODES, NC, DH).transpose(1, 0, 2)
    col = edge_index[1]
    row = edge_index[0]

    k = pl.kernel(
        _gcn_body,
        out_type=jax.ShapeDtypeStruct((N_NODES, D_FEAT), jnp.float32),
        mesh=plsc.VectorSubcoreMesh(core_axis_name="c", subcore_axis_name="s"),
        compiler_params=pltpu.CompilerParams(use_tc_tiling_on_sc=False),
        scratch_types=[
            pltpu.VMEM((EPT,), jnp.int32),
            pltpu.VMEM((EPT,), jnp.float32),
            pltpu.VMEM((CHUNK,), jnp.int32),
            pltpu.VMEM((CHUNK,), jnp.int32),
            pltpu.VMEM((CHUNK, DH), jnp.float32),
            pltpu.VMEM((CHUNK, DH), jnp.float32),
            pltpu.VMEM((RBLK, DH), jnp.float32),
            pltpu.VMEM((RBLK, DH), jnp.float32),
            pltpu.VMEM_SHARED((N_NODES, DH), jnp.float32),
            pltpu.SemaphoreType.DMA,
            pltpu.SemaphoreType.DMA,
            pltpu.SemaphoreType.DMA,
            pltpu.SemaphoreType.DMA,
        ],
    )
    return k(emb_split, col, row, edge_vals)


# speedup vs baseline: 1.4275x; 1.0211x over previous
import jax
import jax.numpy as jnp
from jax import lax
from jax.experimental import pallas as pl
from jax.experimental.pallas import tpu as pltpu, tpu_sc as plsc

N_NODES = 10000
N_EDGES = 160000
D_FEAT = 256

NC = 2
NS = 16
DH = D_FEAT // NC
EPT = N_EDGES // NS
CHUNK = 80
NCHUNK = EPT // CHUNK
RPT = N_NODES // NS
RBLK = 25
NRB = RPT // RBLK


def _gcn_body(emb_hbm, col_hbm, row_hbm, vals_hbm, out_hbm,
              col_v, row_v, valsb0, valsb1, rows0, rows1, ob0, ob1, acc,
              sem0, sem1, osem0, osem1):
    c = lax.axis_index("c")
    s = lax.axis_index("s")

    ebase = s * EPT
    pltpu.async_copy(col_hbm.at[pl.ds(ebase, EPT)], col_v, sem1)
    pltpu.async_copy(row_hbm.at[pl.ds(ebase, EPT)], row_v, sem1)

    def _zset(i, _):
        for g in range(DH // 16):
            ob0[i, pl.ds(g * 16, 16)] = jnp.zeros((16,), jnp.float32)
        return 0
    lax.fori_loop(0, RBLK, _zset, 0)
    for b in range(NRB):
        pltpu.async_copy(ob0, acc.at[pl.ds(s * RPT + b * RBLK, RBLK), :],
                         sem0)
    for b in range(NRB):
        pltpu.make_async_copy(ob0, acc.at[pl.ds(s * RPT, RBLK), :],
                              sem0).wait()
    pltpu.make_async_copy(col_hbm.at[pl.ds(0, EPT)], col_v, sem1).wait()
    pltpu.make_async_copy(row_hbm.at[pl.ds(0, EPT)], row_v, sem1).wait()
    plsc.subcore_barrier()

    emb_c = emb_hbm.at[c]

    def _gather(ch, buf, valsb, sem):
        idx = col_v.at[pl.ds(ch * CHUNK, CHUNK)]
        pltpu.async_copy(emb_c.at[idx], buf, sem)
        pltpu.async_copy(vals_hbm.at[pl.ds(ebase + ch * CHUNK, CHUNK)],
                         valsb, sem)

    def _wait(buf, valsb, sem):
        pltpu.make_async_copy(emb_c.at[col_v.at[pl.ds(0, CHUNK)]], buf,
                              sem).wait()
        pltpu.make_async_copy(vals_hbm.at[pl.ds(0, CHUNK)], valsb,
                              sem).wait()

    def _compute_scatter(ch, buf, valsb):
        for jo in range(0, CHUNK, 16):
            vblk = valsb[pl.ds(jo, 16)]
            for ji in range(16):
                j = jo + ji
                vv = vblk[ji]
                for g in range(DH // 16):
                    sl = pl.ds(g * 16, 16)
                    buf[j, sl] = buf[j, sl] * vv
        pltpu.sync_copy(buf, acc.at[row_v.at[pl.ds(ch * CHUNK, CHUNK)]],
                        add=True)

    _gather(0, rows0, valsb0, sem0)
    _gather(1, rows1, valsb1, sem1)

    def _pair(i, _):
        ch0 = 2 * i
        _wait(rows0, valsb0, sem0)
        _compute_scatter(ch0, rows0, valsb0)
        _gather(ch0 + 2, rows0, valsb0, sem0)
        _wait(rows1, valsb1, sem1)
        _compute_scatter(ch0 + 1, rows1, valsb1)

        @pl.when(i < (NCHUNK - 1) // 2 - 1)
        def _():
            _gather(ch0 + 3, rows1, valsb1, sem1)
        return 0
    lax.fori_loop(0, (NCHUNK - 1) // 2, _pair, 0)

    _wait(rows0, valsb0, sem0)
    _compute_scatter(NCHUNK - 1, rows0, valsb0)
    plsc.subcore_barrier()

    rbase = s * RPT

    def _ain(b, ob, isem):
        pltpu.async_copy(acc.at[pl.ds(rbase + b * RBLK, RBLK), :], ob, isem)

    def _iwait(ob, isem):
        pltpu.make_async_copy(acc.at[pl.ds(rbase, RBLK), :], ob, isem).wait()

    def _aout(b, ob, osem):
        pltpu.async_copy(
            ob,
            out_hbm.at[pl.ds(rbase + b * RBLK, RBLK), pl.ds(c * DH, DH)],
            osem)

    def _owait(ob, osem):
        pltpu.make_async_copy(
            ob, out_hbm.at[pl.ds(rbase, RBLK), pl.ds(c * DH, DH)],
            osem).wait()

    def _lrelu(ob):
        def body(i, _):
            for g in range(DH // 16):
                sl = pl.ds(g * 16, 16)
                x = ob[i, sl]
                ob[i, sl] = jnp.maximum(x, x * 0.5)
            return 0
        lax.fori_loop(0, RBLK, body, 0)

    _ain(0, ob0, sem0)
    _ain(1, ob1, sem1)

    def _dpair(i, _):
        b0 = 2 * i
        _iwait(ob0, sem0)
        _lrelu(ob0)
        _aout(b0, ob0, osem0)
        _iwait(ob1, sem1)
        _lrelu(ob1)
        _aout(b0 + 1, ob1, osem1)
        _owait(ob0, osem0)
        _ain(b0 + 2, ob0, sem0)

        @pl.when(i < NRB // 2 - 1)
        def _():
            _owait(ob1, osem1)
            _ain(b0 + 3, ob1, sem1)
        return 0
    lax.fori_loop(0, NRB // 2, _dpair, 0)

    _iwait(ob0, sem0)
    _lrelu(ob0)
    _aout(NRB - 1, ob0, osem0)
    _owait(ob0, osem0)
    _owait(ob1, osem1)


def kernel(edge_index, edge_vals, embeds):
    emb_split = embeds.reshape(N_NODES, NC, DH).transpose(1, 0, 2)
    col = edge_index[1]
    row = edge_index[0]

    k = pl.kernel(
        _gcn_body,
        out_type=jax.ShapeDtypeStruct((N_NODES, D_FEAT), jnp.float32),
        mesh=plsc.VectorSubcoreMesh(core_axis_name="c", subcore_axis_name="s"),
        compiler_params=pltpu.CompilerParams(use_tc_tiling_on_sc=False),
        scratch_types=[
            pltpu.VMEM((EPT,), jnp.int32),
            pltpu.VMEM((EPT,), jnp.int32),
            pltpu.VMEM((CHUNK,), jnp.float32),
            pltpu.VMEM((CHUNK,), jnp.float32),
            pltpu.VMEM((CHUNK, DH), jnp.float32),
            pltpu.VMEM((CHUNK, DH), jnp.float32),
            pltpu.VMEM((RBLK, DH), jnp.float32),
            pltpu.VMEM((RBLK, DH), jnp.float32),
            pltpu.VMEM_SHARED((N_NODES, DH), jnp.float32),
            pltpu.SemaphoreType.DMA,
            pltpu.SemaphoreType.DMA,
            pltpu.SemaphoreType.DMA,
            pltpu.SemaphoreType.DMA,
        ],
    )
    return k(emb_split, col, row, edge_vals)
